# baseline (speedup 1.0000x reference)
import functools

import jax
import jax.numpy as jnp
from jax import lax
from jax.experimental import pallas as pl
from jax.experimental.pallas import tpu as pltpu
from jax.experimental.pallas import tpu_sc as plsc

N = 10000
D = 128
E = 320000
K = 128
LANES = 16
GROUPS = D // LANES


def _sc_aggregate_fn(nc, ns, chunks_per_worker):
    nw = nc * ns
    epw = chunks_per_worker * K
    rows_per_tile = -(-N // (ns * 8)) * 8
    n_pad = ns * rows_per_tile

    mesh = plsc.VectorSubcoreMesh(core_axis_name="c", subcore_axis_name="s",
                                  num_cores=nc, num_subcores=ns)

    cpw = chunks_per_worker

    @functools.partial(
        pl.kernel,
        out_type=jax.ShapeDtypeStruct((2 * n_pad, D), jnp.float32),
        mesh=mesh,
        scratch_types=[
            pltpu.VMEM((K,), jnp.int32),
            pltpu.VMEM((K,), jnp.int32),
            pltpu.VMEM((K,), jnp.int32),
            pltpu.VMEM((K,), jnp.int32),
            pltpu.VMEM((K,), jnp.float32),
            pltpu.VMEM((K,), jnp.float32),
            pltpu.VMEM((K, D), jnp.float32),
            pltpu.VMEM((K, D), jnp.float32),
            pltpu.VMEM_SHARED((n_pad, D), jnp.float32),
            pltpu.SemaphoreType.DMA,
            pltpu.SemaphoreType.DMA,
            pltpu.SemaphoreType.DMA,
            pltpu.SemaphoreType.DMA,
        ],
    )
    def agg(x_hbm, src_hbm, dst_hbm, w_hbm, out_hbm,
            src_a, src_b, dst_a, dst_b, w_a, w_b,
            rows_a, rows_b, acc, ga, gb, sa, sb):
        srcs = (src_a, src_b)
        dsts = (dst_a, dst_b)
        ws = (w_a, w_b)
        cid = lax.axis_index("c")
        sid = lax.axis_index("s")
        wid = sid * nc + cid
        base = wid * cpw * K

        @pl.loop(0, K)
        def _zero_buf(i):
            for j in range(GROUPS):
                rows_a[i, pl.ds(j * LANES, LANES)] = jnp.zeros((LANES,), jnp.float32)

        row0 = sid * rows_per_tile
        done = 0
        while done < rows_per_tile:
            n = min(K, rows_per_tile - done)
            pltpu.sync_copy(rows_a.at[pl.ds(0, n)], acc.at[pl.ds(row0 + done, n)])
            done += n
        plsc.subcore_barrier()

        def load_idx(c, b):
            off = base + c * K
            pltpu.sync_copy(src_hbm.at[pl.ds(off, K)], srcs[b])
            pltpu.sync_copy(dst_hbm.at[pl.ds(off, K)], dsts[b])
            pltpu.sync_copy(w_hbm.at[pl.ds(off, K)], ws[b])

        def scale(rows_v, b):
            w_ref = ws[b]

            @pl.loop(0, K // LANES)
            def _scale(ii):
                wv = w_ref[pl.ds(ii * LANES, LANES)]
                for l in range(LANES):
                    w = wv[l]
                    i = ii * LANES + l
                    for j in range(GROUPS):
                        sl = pl.ds(j * LANES, LANES)
                        rows_v[i, sl] = rows_v[i, sl] * w

        def gather(b, rows_v, sem):
            pltpu.async_copy(x_hbm.at[srcs[b]], rows_v, sem)

        def scatter(b, rows_v, sem):
            pltpu.async_copy(rows_v, acc.at[dsts[b]], sem, add=True)

        def wait_gather(b, rows_v, sem):
            pltpu.make_async_copy(x_hbm.at[srcs[b]], rows_v, sem).wait()

        def wait_scatter(b, rows_v, sem):
            pltpu.make_async_copy(rows_v, acc.at[dsts[b]], sem).wait()

        load_idx(0, 0)
        gather(0, rows_a, ga)
        load_idx(1, 1)
        gather(1, rows_b, gb)

        @pl.loop(0, cpw, step=2)
        def _chunks(cc):
            wait_gather(0, rows_a, ga)
            scale(rows_a, 0)
            scatter(0, rows_a, sa)

            wait_gather(1, rows_b, gb)
            scale(rows_b, 1)
            scatter(1, rows_b, sb)

            @pl.when(cc + 2 < cpw)
            def _():
                wait_scatter(0, rows_a, sa)
                load_idx(cc + 2, 0)
                gather(0, rows_a, ga)

            @pl.when(cc + 3 < cpw)
            def _():
                wait_scatter(1, rows_b, sb)
                load_idx(cc + 3, 1)
                gather(1, rows_b, gb)

        wait_scatter(0, rows_a, sa)
        wait_scatter(1, rows_b, sb)
        plsc.subcore_barrier()

        pltpu.sync_copy(acc.at[pl.ds(row0, rows_per_tile)],
                        out_hbm.at[pl.ds(cid * n_pad + row0, rows_per_tile)])

    return agg, n_pad


def _dense_kernel(x_ref, p0_ref, p1_ref, w_ref, b_ref, g_ref, be_ref,
                  h_ref, y_ref):
    hi = x_ref[...] + p0_ref[...] + p1_ref[...]
    z = jnp.dot(hi, w_ref[...], preferred_element_type=jnp.float32) + b_ref[...]
    z = jnp.where(z >= 0, z, 0.01 * z)
    m = jnp.mean(z, axis=-1, keepdims=True)
    v = jnp.mean((z - m) ** 2, axis=-1, keepdims=True)
    h = (z - m) * lax.rsqrt(v + 1e-5) * g_ref[...] + be_ref[...]
    h_ref[...] = h
    nrm = jnp.sqrt(jnp.sum(h * h, axis=-1, keepdims=True))
    y_ref[...] = h / jnp.maximum(nrm, 1e-12)


def _dense_layer(x, p0, p1, W, b, g, be):
    blk = 2000
    grid = (N // blk,)
    row_spec = pl.BlockSpec((blk, D), lambda i: (i, 0))
    rep_spec = pl.BlockSpec((1, D), lambda i: (0, 0))
    return pl.pallas_call(
        _dense_kernel,
        grid=grid,
        in_specs=[row_spec, row_spec, row_spec,
                  pl.BlockSpec((D, D), lambda i: (0, 0)),
                  rep_spec, rep_spec, rep_spec],
        out_specs=[row_spec, row_spec],
        out_shape=[jax.ShapeDtypeStruct((N, D), jnp.float32),
                   jax.ShapeDtypeStruct((N, D), jnp.float32)],
    )(x, p0, p1, W, b.reshape(1, D), g.reshape(1, D), be.reshape(1, D))


def kernel(ego_embeddings, edge_index, edge_weight, W1, b1, g1, be1,
           W2, b2, g2, be2):
    info = plsc.get_sparse_core_info()
    nc, ns = info.num_cores, info.num_subcores
    nw = nc * ns
    chunks_per_worker = -(-E // (nw * K * 8)) * 8
    e_pad = nw * chunks_per_worker * K

    src = edge_index[0]
    dst = edge_index[1]
    pad = e_pad - E
    if pad:
        src = jnp.concatenate([src, jnp.zeros((pad,), jnp.int32)])
        dst = jnp.concatenate([dst, jnp.zeros((pad,), jnp.int32)])
        edge_weight = jnp.concatenate([edge_weight, jnp.zeros((pad,), jnp.float32)])

    agg, n_pad = _sc_aggregate_fn(nc, ns, chunks_per_worker)

    def layer(x, W, b, g, be):
        part = agg(x, src, dst, edge_weight)
        return _dense_layer(x, part[:N], part[n_pad:n_pad + N], W, b, g, be)

    h1, y1 = layer(ego_embeddings, W1, b1, g1, be1)
    _, y2 = layer(h1, W2, b2, g2, be2)
    return jnp.concatenate([ego_embeddings, y1, y2], axis=1)


# speedup vs baseline: 1.8838x; 1.8838x over previous
import functools

import jax
import jax.numpy as jnp
from jax import lax
from jax.experimental import pallas as pl
from jax.experimental.pallas import tpu as pltpu
from jax.experimental.pallas import tpu_sc as plsc

N = 10000
D = 128
E = 320000
K = 128
LANES = 16
GROUPS = D // LANES


HALF = D // 2


def _sc_aggregate_fn(nc, ns, cpw0, cpw1):
    rows_per_tile = -(-N // (ns * 8)) * 8
    n_pad = ns * rows_per_tile
    e0 = ns * cpw0 * K

    mesh = plsc.VectorSubcoreMesh(core_axis_name="c", subcore_axis_name="s",
                                  num_cores=nc, num_subcores=ns)

    @functools.partial(
        pl.kernel,
        out_type=jax.ShapeDtypeStruct((2 * n_pad, D), jnp.float32),
        mesh=mesh,
        scratch_types=[
            pltpu.VMEM((K,), jnp.int32),
            pltpu.VMEM((K,), jnp.int32),
            pltpu.VMEM((K,), jnp.int32),
            pltpu.VMEM((K,), jnp.int32),
            pltpu.VMEM((K,), jnp.float32),
            pltpu.VMEM((K,), jnp.float32),
            pltpu.VMEM((K, D), jnp.float32),
            pltpu.VMEM((K, D), jnp.float32),
            pltpu.VMEM_SHARED((n_pad, D), jnp.float32),
            pltpu.SemaphoreType.DMA,
            pltpu.SemaphoreType.DMA,
            pltpu.SemaphoreType.DMA,
            pltpu.SemaphoreType.DMA,
        ],
    )
    def agg(x_hbm, src_hbm, dst_hbm, w_hbm, out_hbm,
            src_a, src_b, dst_a, dst_b, w_a, w_b,
            rows_a, rows_b, acc, ga, gb, sa, sb):
        srcs = (src_a, src_b)
        dsts = (dst_a, dst_b)
        ws = (w_a, w_b)
        cid = lax.axis_index("c")
        sid = lax.axis_index("s")
        my_cpw = lax.select(cid == 0, cpw0, cpw1)
        base = lax.select(cid == 0, sid * (cpw0 * K), e0 + sid * (cpw1 * K))
        row0 = sid * rows_per_tile

        @pl.loop(0, K)
        def _zero_buf(i):
            for j in range(GROUPS):
                rows_a[i, pl.ds(j * LANES, LANES)] = jnp.zeros((LANES,), jnp.float32)

        done = 0
        while done < rows_per_tile:
            n = min(K, rows_per_tile - done)
            pltpu.sync_copy(rows_a.at[pl.ds(0, n)], acc.at[pl.ds(row0 + done, n)])
            done += n
        plsc.subcore_barrier()

        def load_idx(c, b):
            off = base + c * K
            pltpu.sync_copy(src_hbm.at[pl.ds(off, K)], srcs[b])
            pltpu.sync_copy(dst_hbm.at[pl.ds(off, K)], dsts[b])
            pltpu.sync_copy(w_hbm.at[pl.ds(off, K)], ws[b])

        def scale(rows_v, b):
            w_ref = ws[b]

            @pl.loop(0, K // LANES)
            def _scale(ii):
                wv = w_ref[pl.ds(ii * LANES, LANES)]
                for l in range(LANES):
                    w = wv[l]
                    i = ii * LANES + l
                    for j in range(GROUPS):
                        sl = pl.ds(j * LANES, LANES)
                        rows_v[i, sl] = rows_v[i, sl] * w

        def gather(b, rows_v, sem):
            pltpu.async_copy(x_hbm.at[srcs[b]], rows_v, sem)

        def scatter(b, rows_v, sem):
            pltpu.async_copy(rows_v, acc.at[dsts[b]], sem, add=True)

        def wait_gather(b, rows_v, sem):
            pltpu.make_async_copy(x_hbm.at[srcs[b]], rows_v, sem).wait()

        def wait_scatter(b, rows_v, sem):
            pltpu.make_async_copy(rows_v, acc.at[dsts[b]], sem).wait()

        load_idx(0, 0)
        gather(0, rows_a, ga)
        load_idx(1, 1)
        gather(1, rows_b, gb)

        @pl.loop(0, my_cpw, step=2)
        def _chunks(cc):
            wait_gather(0, rows_a, ga)
            scale(rows_a, 0)
            scatter(0, rows_a, sa)

            wait_gather(1, rows_b, gb)
            scale(rows_b, 1)
            scatter(1, rows_b, sb)

            @pl.when(cc + 2 < my_cpw)
            def _():
                wait_scatter(0, rows_a, sa)
                load_idx(cc + 2, 0)
                gather(0, rows_a, ga)

            @pl.when(cc + 3 < my_cpw)
            def _():
                wait_scatter(1, rows_b, sb)
                load_idx(cc + 3, 1)
                gather(1, rows_b, gb)

        wait_scatter(0, rows_a, sa)
        wait_scatter(1, rows_b, sb)
        plsc.subcore_barrier()

        pltpu.sync_copy(acc.at[pl.ds(row0, rows_per_tile)],
                        out_hbm.at[pl.ds(cid * n_pad + row0, rows_per_tile)])

    return agg, n_pad


def _dense_kernel(x_ref, p0_ref, p1_ref, w_ref, b_ref, g_ref, be_ref,
                  h_ref, y_ref):
    hi = x_ref[...] + p0_ref[...] + p1_ref[...]
    z = jnp.dot(hi, w_ref[...], preferred_element_type=jnp.float32) + b_ref[...]
    z = jnp.where(z >= 0, z, 0.01 * z)
    m = jnp.mean(z, axis=-1, keepdims=True)
    v = jnp.mean((z - m) ** 2, axis=-1, keepdims=True)
    h = (z - m) * lax.rsqrt(v + 1e-5) * g_ref[...] + be_ref[...]
    h_ref[...] = h
    nrm = jnp.sqrt(jnp.sum(h * h, axis=-1, keepdims=True))
    y_ref[...] = h / jnp.maximum(nrm, 1e-12)


def _dense_layer(x, p0, p1, W, b, g, be):
    blk = 2000
    grid = (N // blk,)
    row_spec = pl.BlockSpec((blk, D), lambda i: (i, 0))
    rep_spec = pl.BlockSpec((1, D), lambda i: (0, 0))
    return pl.pallas_call(
        _dense_kernel,
        grid=grid,
        in_specs=[row_spec, row_spec, row_spec,
                  pl.BlockSpec((D, D), lambda i: (0, 0)),
                  rep_spec, rep_spec, rep_spec],
        out_specs=[row_spec, row_spec],
        out_shape=[jax.ShapeDtypeStruct((N, D), jnp.float32),
                   jax.ShapeDtypeStruct((N, D), jnp.float32)],
    )(x, p0, p1, W, b.reshape(1, D), g.reshape(1, D), be.reshape(1, D))


def kernel(ego_embeddings, edge_index, edge_weight, W1, b1, g1, be1,
           W2, b2, g2, be2):
    info = plsc.get_sparse_core_info()
    nc, ns = info.num_cores, info.num_subcores
    tot_cpt = -(-E // (ns * K * 2)) * 2
    cpw0 = int(round(tot_cpt * 0.73 / 2)) * 2
    cpw1 = tot_cpt - cpw0
    e_pad = ns * tot_cpt * K

    src = edge_index[0]
    dst = edge_index[1]
    pad = e_pad - E
    if pad:
        src = jnp.concatenate([src, jnp.zeros((pad,), jnp.int32)])
        dst = jnp.concatenate([dst, jnp.zeros((pad,), jnp.int32)])
        edge_weight = jnp.concatenate([edge_weight, jnp.zeros((pad,), jnp.float32)])

    agg, n_pad = _sc_aggregate_fn(nc, ns, cpw0, cpw1)

    def layer(x, W, b, g, be):
        part = agg(x, src, dst, edge_weight)
        return _dense_layer(x, part[:N], part[n_pad:n_pad + N], W, b, g, be)

    h1, y1 = layer(ego_embeddings, W1, b1, g1, be1)
    _, y2 = layer(h1, W2, b2, g2, be2)
    return jnp.concatenate([ego_embeddings, y1, y2], axis=1)


# speedup vs baseline: 1.9987x; 1.0610x over previous
import functools

import jax
import jax.numpy as jnp
from jax import lax
from jax.experimental import pallas as pl
from jax.experimental.pallas import tpu as pltpu
from jax.experimental.pallas import tpu_sc as plsc

N = 10000
D = 128
E = 320000
K = 112
LANES = 16
GROUPS = D // LANES
PAIRS = D // 32

_PERM = [32 * g + (k // 2 if k % 2 == 0 else 16 + k // 2)
         for g in range(PAIRS) for k in range(32)]


HALF = D // 2


def _sc_aggregate_fn(nc, ns, cpw0, cpw1):
    rows_per_tile = -(-N // (ns * 8)) * 8
    n_pad = ns * rows_per_tile
    e0 = ns * cpw0 * K

    mesh = plsc.VectorSubcoreMesh(core_axis_name="c", subcore_axis_name="s",
                                  num_cores=nc, num_subcores=ns)

    @functools.partial(
        pl.kernel,
        out_type=jax.ShapeDtypeStruct((2 * n_pad, D), jnp.float32),
        mesh=mesh,
        scratch_types=[
            pltpu.VMEM((K,), jnp.int32),
            pltpu.VMEM((K,), jnp.int32),
            pltpu.VMEM((K,), jnp.int32),
            pltpu.VMEM((K,), jnp.int32),
            pltpu.VMEM((K,), jnp.int32),
            pltpu.VMEM((K,), jnp.int32),
            pltpu.VMEM((K,), jnp.float32),
            pltpu.VMEM((K,), jnp.float32),
            pltpu.VMEM((K,), jnp.float32),
            pltpu.VMEM((K, D), jnp.float32),
            pltpu.VMEM((K, D), jnp.float32),
            pltpu.VMEM((K, D), jnp.float32),
            pltpu.VMEM_SHARED((n_pad, D), jnp.float32),
            pltpu.SemaphoreType.DMA,
            pltpu.SemaphoreType.DMA,
            pltpu.SemaphoreType.DMA,
            pltpu.SemaphoreType.DMA,
            pltpu.SemaphoreType.DMA,
            pltpu.SemaphoreType.DMA,
        ],
    )
    def agg(x_hbm, src_hbm, dst_hbm, w_hbm, out_hbm,
            src_0, src_1, src_2, dst_0, dst_1, dst_2, w_0, w_1, w_2,
            rows_0, rows_1, rows_2, acc, g0, g1, g2, s0, s1, s2):
        srcs = (src_0, src_1, src_2)
        dsts = (dst_0, dst_1, dst_2)
        ws = (w_0, w_1, w_2)
        rows = (rows_0, rows_1, rows_2)
        gsem = (g0, g1, g2)
        ssem = (s0, s1, s2)
        cid = lax.axis_index("c")
        sid = lax.axis_index("s")
        my_cpw = lax.select(cid == 0, cpw0, cpw1)
        base = lax.select(cid == 0, sid * (cpw0 * K), e0 + sid * (cpw1 * K))
        row0 = sid * rows_per_tile

        @pl.loop(0, K)
        def _zero_buf(i):
            for j in range(GROUPS):
                rows_0[i, pl.ds(j * LANES, LANES)] = jnp.zeros((LANES,), jnp.float32)

        done = 0
        while done < rows_per_tile:
            n = min(K, rows_per_tile - done)
            pltpu.sync_copy(rows_0.at[pl.ds(0, n)], acc.at[pl.ds(row0 + done, n)])
            done += n
        plsc.subcore_barrier()

        def load_idx(c, b):
            off = base + c * K
            pltpu.sync_copy(src_hbm.at[pl.ds(off, K)], srcs[b])
            pltpu.sync_copy(dst_hbm.at[pl.ds(off, K)], dsts[b])
            pltpu.sync_copy(w_hbm.at[pl.ds(off, K)], ws[b])

        def scale(t):
            w_ref = ws[t]
            rows_v = rows[t]

            @pl.loop(0, K // LANES, unroll=2)
            def _scale(ii):
                wv = w_ref[pl.ds(ii * LANES, LANES)]
                for l in range(LANES):
                    w = wv[l]
                    i = ii * LANES + l
                    for j in range(GROUPS):
                        sl = pl.ds(j * LANES, LANES)
                        rows_v[i, sl] = rows_v[i, sl] * w

        def gather(t):
            pltpu.async_copy(x_hbm.at[srcs[t]], rows[t], gsem[t])

        def scatter(t):
            pltpu.async_copy(rows[t], acc.at[dsts[t]], ssem[t], add=True)

        def wait_gather(t):
            pltpu.make_async_copy(x_hbm.at[srcs[t]], rows[t], gsem[t]).wait()

        def wait_scatter(t):
            pltpu.make_async_copy(rows[t], acc.at[dsts[t]], ssem[t]).wait()

        for t in range(3):
            load_idx(t, t)
            gather(t)

        @pl.loop(0, my_cpw, step=3)
        def _chunks(cc):
            for t in range(3):
                wait_gather(t)
                scale(t)
                scatter(t)

            for t in range(3):
                @pl.when(cc + t + 3 < my_cpw)
                def _(t=t):
                    wait_scatter(t)
                    load_idx(cc + t + 3, t)
                    gather(t)

        for t in range(3):
            wait_scatter(t)
        plsc.subcore_barrier()

        pltpu.sync_copy(acc.at[pl.ds(row0, rows_per_tile)],
                        out_hbm.at[pl.ds(cid * n_pad + row0, rows_per_tile)])

    return agg, n_pad


def _dense_kernel(x_ref, p0_ref, p1_ref, w_ref, b_ref, g_ref, be_ref,
                  h_ref, y_ref):
    hi = x_ref[...] + p0_ref[...] + p1_ref[...]
    z = jnp.dot(hi, w_ref[...], preferred_element_type=jnp.float32) + b_ref[...]
    z = jnp.where(z >= 0, z, 0.01 * z)
    m = jnp.mean(z, axis=-1, keepdims=True)
    v = jnp.mean((z - m) ** 2, axis=-1, keepdims=True)
    h = (z - m) * lax.rsqrt(v + 1e-5) * g_ref[...] + be_ref[...]
    h_ref[...] = h
    nrm = jnp.sqrt(jnp.sum(h * h, axis=-1, keepdims=True))
    y_ref[...] = h / jnp.maximum(nrm, 1e-12)


def _dense_layer(x, p0, p1, W, b, g, be):
    blk = 2000
    grid = (N // blk,)
    row_spec = pl.BlockSpec((blk, D), lambda i: (i, 0))
    rep_spec = pl.BlockSpec((1, D), lambda i: (0, 0))
    return pl.pallas_call(
        _dense_kernel,
        grid=grid,
        in_specs=[row_spec, row_spec, row_spec,
                  pl.BlockSpec((D, D), lambda i: (0, 0)),
                  rep_spec, rep_spec, rep_spec],
        out_specs=[row_spec, row_spec],
        out_shape=[jax.ShapeDtypeStruct((N, D), jnp.float32),
                   jax.ShapeDtypeStruct((N, D), jnp.float32)],
    )(x, p0, p1, W, b.reshape(1, D), g.reshape(1, D), be.reshape(1, D))


def kernel(ego_embeddings, edge_index, edge_weight, W1, b1, g1, be1,
           W2, b2, g2, be2):
    info = plsc.get_sparse_core_info()
    nc, ns = info.num_cores, info.num_subcores
    tot_cpt = -(-E // (ns * K * 3)) * 3
    cpw0 = int(round(tot_cpt * 0.73 / 3)) * 3
    cpw1 = tot_cpt - cpw0
    e_pad = ns * tot_cpt * K

    src = edge_index[0]
    dst = edge_index[1]
    pad = e_pad - E
    if pad:
        src = jnp.concatenate([src, jnp.zeros((pad,), jnp.int32)])
        dst = jnp.concatenate([dst, jnp.zeros((pad,), jnp.int32)])
        edge_weight = jnp.concatenate([edge_weight, jnp.zeros((pad,), jnp.float32)])

    agg, n_pad = _sc_aggregate_fn(nc, ns, cpw0, cpw1)

    def layer(x, W, b, g, be):
        part = agg(x, src, dst, edge_weight)
        return _dense_layer(x, part[:N], part[n_pad:n_pad + N], W, b, g, be)

    h1, y1 = layer(ego_embeddings, W1, b1, g1, be1)
    _, y2 = layer(h1, W2, b2, g2, be2)
    return jnp.concatenate([ego_embeddings, y1, y2], axis=1)


# speedup vs baseline: 2.0726x; 1.0369x over previous
import functools

import jax
import jax.numpy as jnp
from jax import lax
from jax.experimental import pallas as pl
from jax.experimental.pallas import tpu as pltpu
from jax.experimental.pallas import tpu_sc as plsc

N = 10000
D = 128
E = 320000
K = 112
LANES = 16
GROUPS = D // LANES
PAIRS = D // 32

_PERM = [32 * g + (k // 2 if k % 2 == 0 else 16 + k // 2)
         for g in range(PAIRS) for k in range(32)]


HALF = D // 2


def _sc_aggregate_fn(nc, ns, cpw0, cpw1):
    rows_per_tile = -(-N // (ns * 8)) * 8
    n_pad = ns * rows_per_tile
    e0 = ns * cpw0 * K

    mesh = plsc.VectorSubcoreMesh(core_axis_name="c", subcore_axis_name="s",
                                  num_cores=nc, num_subcores=ns)

    @functools.partial(
        pl.kernel,
        out_type=jax.ShapeDtypeStruct((2 * n_pad, D), jnp.float32),
        mesh=mesh,
        scratch_types=[
            pltpu.VMEM((K,), jnp.int32),
            pltpu.VMEM((K,), jnp.int32),
            pltpu.VMEM((K,), jnp.int32),
            pltpu.VMEM((K,), jnp.int32),
            pltpu.VMEM((K,), jnp.int32),
            pltpu.VMEM((K,), jnp.int32),
            pltpu.VMEM((K,), jnp.float32),
            pltpu.VMEM((K,), jnp.float32),
            pltpu.VMEM((K,), jnp.float32),
            pltpu.VMEM((K, D), jnp.float32),
            pltpu.VMEM((K, D), jnp.float32),
            pltpu.VMEM((K, D), jnp.float32),
            pltpu.VMEM_SHARED((n_pad, D), jnp.float32),
            pltpu.SemaphoreType.DMA,
            pltpu.SemaphoreType.DMA,
            pltpu.SemaphoreType.DMA,
            pltpu.SemaphoreType.DMA,
            pltpu.SemaphoreType.DMA,
            pltpu.SemaphoreType.DMA,
        ],
    )
    def agg(x_hbm, src_hbm, dst_hbm, w_hbm, out_hbm,
            src_0, src_1, src_2, dst_0, dst_1, dst_2, w_0, w_1, w_2,
            rows_0, rows_1, rows_2, acc, g0, g1, g2, s0, s1, s2):
        srcs = (src_0, src_1, src_2)
        dsts = (dst_0, dst_1, dst_2)
        ws = (w_0, w_1, w_2)
        rows = (rows_0, rows_1, rows_2)
        gsem = (g0, g1, g2)
        ssem = (s0, s1, s2)
        cid = lax.axis_index("c")
        sid = lax.axis_index("s")
        my_cpw = lax.select(cid == 0, cpw0, cpw1)
        base = lax.select(cid == 0, sid * (cpw0 * K), e0 + sid * (cpw1 * K))
        row0 = sid * rows_per_tile

        @pl.loop(0, K)
        def _zero_buf(i):
            for j in range(GROUPS):
                rows_0[i, pl.ds(j * LANES, LANES)] = jnp.zeros((LANES,), jnp.float32)

        done = 0
        while done < rows_per_tile:
            n = min(K, rows_per_tile - done)
            pltpu.sync_copy(rows_0.at[pl.ds(0, n)], acc.at[pl.ds(row0 + done, n)])
            done += n
        plsc.subcore_barrier()

        def load_idx(c, b):
            off = base + c * K
            pltpu.sync_copy(src_hbm.at[pl.ds(off, K)], srcs[b])
            pltpu.sync_copy(dst_hbm.at[pl.ds(off, K)], dsts[b])
            pltpu.sync_copy(w_hbm.at[pl.ds(off, K)], ws[b])

        def scale(t):
            w_ref = ws[t]
            rows_v = rows[t]

            @pl.loop(0, K // LANES, unroll=2)
            def _scale(ii):
                wv = w_ref[pl.ds(ii * LANES, LANES)]
                for l in range(LANES):
                    w = wv[l]
                    i = ii * LANES + l
                    for j in range(GROUPS):
                        sl = pl.ds(j * LANES, LANES)
                        rows_v[i, sl] = rows_v[i, sl] * w

        def gather(t):
            pltpu.async_copy(x_hbm.at[srcs[t]], rows[t], gsem[t])

        def scatter(t):
            pltpu.async_copy(rows[t], acc.at[dsts[t]], ssem[t], add=True)

        def wait_gather(t):
            pltpu.make_async_copy(x_hbm.at[srcs[t]], rows[t], gsem[t]).wait()

        def wait_scatter(t):
            pltpu.make_async_copy(rows[t], acc.at[dsts[t]], ssem[t]).wait()

        for t in range(3):
            load_idx(t, t)
            gather(t)

        @pl.loop(0, my_cpw, step=3)
        def _chunks(cc):
            for t in range(3):
                wait_gather(t)
                scale(t)
                scatter(t)

            for t in range(3):
                @pl.when(cc + t + 3 < my_cpw)
                def _(t=t):
                    wait_scatter(t)
                    load_idx(cc + t + 3, t)
                    gather(t)

        for t in range(3):
            wait_scatter(t)
        plsc.subcore_barrier()

        pltpu.sync_copy(acc.at[pl.ds(row0, rows_per_tile)],
                        out_hbm.at[pl.ds(cid * n_pad + row0, rows_per_tile)])

    return agg, n_pad


def _dense_kernel(x_ref, p0_ref, p1_ref, w_ref, b_ref, g_ref, be_ref,
                  h_ref, y_ref):
    hi = x_ref[...] + p0_ref[...] + p1_ref[...]
    z = jnp.dot(hi, w_ref[...], preferred_element_type=jnp.float32) + b_ref[...]
    z = jnp.where(z >= 0, z, 0.01 * z)
    m = jnp.mean(z, axis=-1, keepdims=True)
    v = jnp.mean((z - m) ** 2, axis=-1, keepdims=True)
    h = (z - m) * lax.rsqrt(v + 1e-5) * g_ref[...] + be_ref[...]
    h_ref[...] = h
    nrm = jnp.sqrt(jnp.sum(h * h, axis=-1, keepdims=True))
    y_ref[...] = h / jnp.maximum(nrm, 1e-12)


def _dense_layer(x, p0, p1, W, b, g, be):
    blk = 2000
    grid = (N // blk,)
    row_spec = pl.BlockSpec((blk, D), lambda i: (i, 0))
    rep_spec = pl.BlockSpec((1, D), lambda i: (0, 0))
    return pl.pallas_call(
        _dense_kernel,
        grid=grid,
        in_specs=[row_spec, row_spec, row_spec,
                  pl.BlockSpec((D, D), lambda i: (0, 0)),
                  rep_spec, rep_spec, rep_spec],
        out_specs=[row_spec, row_spec],
        out_shape=[jax.ShapeDtypeStruct((N, D), jnp.float32),
                   jax.ShapeDtypeStruct((N, D), jnp.float32)],
    )(x, p0, p1, W, b.reshape(1, D), g.reshape(1, D), be.reshape(1, D))


def kernel(ego_embeddings, edge_index, edge_weight, W1, b1, g1, be1,
           W2, b2, g2, be2):
    info = plsc.get_sparse_core_info()
    nc, ns = info.num_cores, info.num_subcores
    tot_cpt = -(-E // (ns * K * 3)) * 3
    cpw0 = int(round(tot_cpt * 0.70 / 3)) * 3
    cpw1 = tot_cpt - cpw0
    e_pad = ns * tot_cpt * K

    src = edge_index[0]
    dst = edge_index[1]
    pad = e_pad - E
    if pad:
        src = jnp.concatenate([src, jnp.zeros((pad,), jnp.int32)])
        dst = jnp.concatenate([dst, jnp.zeros((pad,), jnp.int32)])
        edge_weight = jnp.concatenate([edge_weight, jnp.zeros((pad,), jnp.float32)])

    agg, n_pad = _sc_aggregate_fn(nc, ns, cpw0, cpw1)

    def layer(x, W, b, g, be):
        part = agg(x, src, dst, edge_weight)
        return _dense_layer(x, part[:N], part[n_pad:n_pad + N], W, b, g, be)

    h1, y1 = layer(ego_embeddings, W1, b1, g1, be1)
    _, y2 = layer(h1, W2, b2, g2, be2)
    return jnp.concatenate([ego_embeddings, y1, y2], axis=1)


# speedup vs baseline: 2.1840x; 1.0538x over previous
import functools

import jax
import jax.numpy as jnp
from jax import lax
from jax.experimental import pallas as pl
from jax.experimental.pallas import tpu as pltpu
from jax.experimental.pallas import tpu_sc as plsc

N = 10000
D = 128
E = 320000
K = 120
LANES = 16
GROUPS = D // LANES
PAIRS = D // 32

_PERM = [32 * g + (k // 2 if k % 2 == 0 else 16 + k // 2)
         for g in range(PAIRS) for k in range(32)]


HALF = D // 2


def _sc_aggregate_fn(nc, ns, cpw0, cpw1):
    rows_per_tile = -(-N // (ns * 8)) * 8
    n_pad = ns * rows_per_tile
    e0 = ns * cpw0 * K

    mesh = plsc.VectorSubcoreMesh(core_axis_name="c", subcore_axis_name="s",
                                  num_cores=nc, num_subcores=ns)

    @functools.partial(
        pl.kernel,
        out_type=jax.ShapeDtypeStruct((2 * n_pad, D), jnp.float32),
        mesh=mesh,
        scratch_types=[
            pltpu.VMEM((K,), jnp.int32),
            pltpu.VMEM((K,), jnp.int32),
            pltpu.VMEM((K,), jnp.int32),
            pltpu.VMEM((K,), jnp.int32),
            pltpu.VMEM((K,), jnp.int32),
            pltpu.VMEM((K,), jnp.int32),
            pltpu.VMEM((K,), jnp.float32),
            pltpu.VMEM((K,), jnp.float32),
            pltpu.VMEM((K,), jnp.float32),
            pltpu.VMEM((K, D), jnp.float32),
            pltpu.VMEM((K, D), jnp.float32),
            pltpu.VMEM((K, D), jnp.float32),
            pltpu.VMEM_SHARED((n_pad, D), jnp.float32),
            pltpu.SemaphoreType.DMA,
            pltpu.SemaphoreType.DMA,
            pltpu.SemaphoreType.DMA,
            pltpu.SemaphoreType.DMA,
            pltpu.SemaphoreType.DMA,
            pltpu.SemaphoreType.DMA,
        ],
    )
    def agg(x_hbm, src_hbm, dst_hbm, w_hbm, out_hbm,
            src_0, src_1, src_2, dst_0, dst_1, dst_2, w_0, w_1, w_2,
            rows_0, rows_1, rows_2, acc, g0, g1, g2, s0, s1, s2):
        srcs = (src_0, src_1, src_2)
        dsts = (dst_0, dst_1, dst_2)
        ws = (w_0, w_1, w_2)
        rows = (rows_0, rows_1, rows_2)
        gsem = (g0, g1, g2)
        ssem = (s0, s1, s2)
        cid = lax.axis_index("c")
        sid = lax.axis_index("s")
        my_cpw = lax.select(cid == 0, cpw0, cpw1)
        base = lax.select(cid == 0, sid * (cpw0 * K), e0 + sid * (cpw1 * K))
        row0 = sid * rows_per_tile

        @pl.loop(0, K)
        def _zero_buf(i):
            for j in range(GROUPS):
                rows_0[i, pl.ds(j * LANES, LANES)] = jnp.zeros((LANES,), jnp.float32)

        done = 0
        while done < rows_per_tile:
            n = min(K, rows_per_tile - done)
            pltpu.sync_copy(rows_0.at[pl.ds(0, n)], acc.at[pl.ds(row0 + done, n)])
            done += n
        plsc.subcore_barrier()

        def load_idx(c, b):
            off = base + c * K
            pltpu.sync_copy(src_hbm.at[pl.ds(off, K)], srcs[b])
            pltpu.sync_copy(dst_hbm.at[pl.ds(off, K)], dsts[b])
            pltpu.sync_copy(w_hbm.at[pl.ds(off, K)], ws[b])

        def scale(t):
            w_ref = ws[t]
            rows_v = rows[t]

            @pl.loop(0, K // LANES, unroll=2)
            def _scale(ii):
                wv = w_ref[pl.ds(ii * LANES, LANES)]
                for l in range(LANES):
                    w = wv[l]
                    i = ii * LANES + l
                    for j in range(GROUPS):
                        sl = pl.ds(j * LANES, LANES)
                        rows_v[i, sl] = rows_v[i, sl] * w

        def gather(t):
            pltpu.async_copy(x_hbm.at[srcs[t]], rows[t], gsem[t])

        def scatter(t):
            pltpu.async_copy(rows[t], acc.at[dsts[t]], ssem[t], add=True)

        def wait_gather(t):
            pltpu.make_async_copy(x_hbm.at[srcs[t]], rows[t], gsem[t]).wait()

        def wait_scatter(t):
            pltpu.make_async_copy(rows[t], acc.at[dsts[t]], ssem[t]).wait()

        for t in range(3):
            load_idx(t, t)
            gather(t)

        @pl.loop(0, my_cpw, step=3)
        def _chunks(cc):
            for t in range(3):
                wait_gather(t)
                scale(t)
                scatter(t)

            for t in range(3):
                @pl.when(cc + t + 3 < my_cpw)
                def _(t=t):
                    wait_scatter(t)
                    load_idx(cc + t + 3, t)
                    gather(t)

        for t in range(3):
            wait_scatter(t)
        plsc.subcore_barrier()

        pltpu.sync_copy(acc.at[pl.ds(row0, rows_per_tile)],
                        out_hbm.at[pl.ds(cid * n_pad + row0, rows_per_tile)])

    return agg, n_pad


def _dense_kernel(x_ref, p0_ref, p1_ref, w_ref, b_ref, g_ref, be_ref,
                  h_ref, y_ref):
    hi = x_ref[...] + p0_ref[...] + p1_ref[...]
    z = jnp.dot(hi, w_ref[...], preferred_element_type=jnp.float32) + b_ref[...]
    z = jnp.where(z >= 0, z, 0.01 * z)
    m = jnp.mean(z, axis=-1, keepdims=True)
    v = jnp.mean((z - m) ** 2, axis=-1, keepdims=True)
    h = (z - m) * lax.rsqrt(v + 1e-5) * g_ref[...] + be_ref[...]
    h_ref[...] = h
    nrm = jnp.sqrt(jnp.sum(h * h, axis=-1, keepdims=True))
    y_ref[...] = h / jnp.maximum(nrm, 1e-12)


def _dense_layer(x, p0, p1, W, b, g, be):
    blk = 2000
    grid = (N // blk,)
    row_spec = pl.BlockSpec((blk, D), lambda i: (i, 0))
    rep_spec = pl.BlockSpec((1, D), lambda i: (0, 0))
    return pl.pallas_call(
        _dense_kernel,
        grid=grid,
        in_specs=[row_spec, row_spec, row_spec,
                  pl.BlockSpec((D, D), lambda i: (0, 0)),
                  rep_spec, rep_spec, rep_spec],
        out_specs=[row_spec, row_spec],
        out_shape=[jax.ShapeDtypeStruct((N, D), jnp.float32),
                   jax.ShapeDtypeStruct((N, D), jnp.float32)],
    )(x, p0, p1, W, b.reshape(1, D), g.reshape(1, D), be.reshape(1, D))


def kernel(ego_embeddings, edge_index, edge_weight, W1, b1, g1, be1,
           W2, b2, g2, be2):
    info = plsc.get_sparse_core_info()
    nc, ns = info.num_cores, info.num_subcores
    tot_cpt = -(-E // (ns * K * 3)) * 3
    cpw0 = int(round(tot_cpt * 0.68 / 3)) * 3
    cpw1 = tot_cpt - cpw0
    e_pad = ns * tot_cpt * K

    src = edge_index[0]
    dst = edge_index[1]
    pad = e_pad - E
    if pad:
        src = jnp.concatenate([src, jnp.zeros((pad,), jnp.int32)])
        dst = jnp.concatenate([dst, jnp.zeros((pad,), jnp.int32)])
        edge_weight = jnp.concatenate([edge_weight, jnp.zeros((pad,), jnp.float32)])

    agg, n_pad = _sc_aggregate_fn(nc, ns, cpw0, cpw1)

    def layer(x, W, b, g, be):
        part = agg(x, src, dst, edge_weight)
        return _dense_layer(x, part[:N], part[n_pad:n_pad + N], W, b, g, be)

    h1, y1 = layer(ego_embeddings, W1, b1, g1, be1)
    _, y2 = layer(h1, W2, b2, g2, be2)
    return jnp.concatenate([ego_embeddings, y1, y2], axis=1)
